# baseline (speedup 1.0000x reference)
import math

import jax
import jax.numpy as jnp
import numpy as np
from jax.experimental import pallas as pl
from jax.experimental.pallas import tpu as pltpu

IN_FEATURES = 1024
LEAF_WIDTH = 64
OUT_FEATURES = 1024
DEPTH = 4
N_LEAVES = 2 ** DEPTH
N_NODES = N_LEAVES - 1

BM = 512


def _routing_mats():
    P = np.zeros((N_NODES, N_LEAVES), np.float32)
    N = np.zeros((N_NODES, N_LEAVES), np.float32)
    for l in range(N_LEAVES):
        for d in range(DEPTH):
            g = l >> (DEPTH - 1 - d)
            node = (2 ** d - 1) + (g >> 1)
            if g & 1:
                P[node, l] = 1.0
            else:
                N[node, l] = 1.0
    E = np.kron(np.eye(N_LEAVES, dtype=np.float32), np.ones((1, LEAF_WIDTH), np.float32))
    return jnp.asarray(P), jnp.asarray(N), jnp.asarray(E)


def _fff_block(x_ref, nwT_ref, nb_ref, P_ref, N_ref, E_ref,
               W1_ref, b1_ref, W2_ref, b2_ref, o_ref):
    xb = x_ref[...]
    z = jnp.dot(xb, nwT_ref[...], preferred_element_type=jnp.float32) + nb_ref[...]
    ls_p = jax.nn.log_sigmoid(z)
    ls_n = jax.nn.log_sigmoid(-z)
    m = jnp.exp(jnp.dot(ls_p, P_ref[...], preferred_element_type=jnp.float32)
                + jnp.dot(ls_n, N_ref[...], preferred_element_type=jnp.float32))
    me = jnp.dot(m, E_ref[...], preferred_element_type=jnp.float32)
    h = jnp.dot(xb, W1_ref[...], preferred_element_type=jnp.float32) + b1_ref[...]
    g = (jnp.maximum(h, 0.0) * me).astype(jnp.bfloat16)
    o_ref[...] = (jnp.dot(g, W2_ref[...], preferred_element_type=jnp.float32)
                  + jnp.dot(m, b2_ref[...], preferred_element_type=jnp.float32))


def kernel(x, node_weights, node_biases, w1s, b1s, w2s, b2s):
    B = x.shape[0]
    P, N, E = _routing_mats()
    xb16 = x.astype(jnp.bfloat16)
    nwT = node_weights.T.astype(jnp.bfloat16)
    nb = node_biases.reshape(1, N_NODES)
    W1 = jnp.transpose(w1s, (1, 0, 2)).reshape(
        IN_FEATURES, N_LEAVES * LEAF_WIDTH).astype(jnp.bfloat16)
    b1 = b1s.reshape(1, N_LEAVES * LEAF_WIDTH)
    W2 = w2s.reshape(N_LEAVES * LEAF_WIDTH, OUT_FEATURES).astype(jnp.bfloat16)

    grid = (B // BM,)
    full = lambda a: pl.BlockSpec(a.shape, lambda i: (0,) * a.ndim)
    out = pl.pallas_call(
        _fff_block,
        grid=grid,
        in_specs=[
            pl.BlockSpec((BM, IN_FEATURES), lambda i: (i, 0)),
            full(nwT), full(nb), full(P), full(N), full(E),
            full(W1), full(b1), full(W2), full(b2s),
        ],
        out_specs=pl.BlockSpec((BM, OUT_FEATURES), lambda i: (i, 0)),
        out_shape=jax.ShapeDtypeStruct((B, OUT_FEATURES), jnp.float32),
        compiler_params=pltpu.CompilerParams(
            dimension_semantics=("parallel",),
        ),
    )(xb16, nwT, nb, P, N, E, W1, b1, W2, b2s)
    return out


# speedup vs baseline: 1.2187x; 1.2187x over previous
import math

import jax
import jax.numpy as jnp
import numpy as np
from jax.experimental import pallas as pl
from jax.experimental.pallas import tpu as pltpu

IN_FEATURES = 1024
LEAF_WIDTH = 64
OUT_FEATURES = 1024
DEPTH = 4
N_LEAVES = 2 ** DEPTH
N_NODES = N_LEAVES - 1

BM = 512


def _routing_mats():
    P = np.zeros((N_NODES, N_LEAVES), np.float32)
    N = np.zeros((N_NODES, N_LEAVES), np.float32)
    for l in range(N_LEAVES):
        for d in range(DEPTH):
            g = l >> (DEPTH - 1 - d)
            node = (2 ** d - 1) + (g >> 1)
            if g & 1:
                P[node, l] = 1.0
            else:
                N[node, l] = 1.0
    E = np.kron(np.eye(N_LEAVES, dtype=np.float32), np.ones((1, LEAF_WIDTH), np.float32))
    return jnp.asarray(P), jnp.asarray(N), jnp.asarray(E)


def _fff_block(x_ref, nwT_ref, nb_ref, P_ref, N_ref, E_ref,
               W1_ref, b1_ref, W2_ref, b2_ref, o_ref):
    xb = x_ref[...]
    z = jnp.dot(xb, nwT_ref[...], preferred_element_type=jnp.float32) + nb_ref[...]
    ls_p = jax.nn.log_sigmoid(z)
    ls_n = jax.nn.log_sigmoid(-z)
    m = jnp.exp(jnp.dot(ls_p, P_ref[...], preferred_element_type=jnp.float32)
                + jnp.dot(ls_n, N_ref[...], preferred_element_type=jnp.float32))
    me = jnp.dot(m, E_ref[...], preferred_element_type=jnp.float32)
    h = jnp.dot(xb, W1_ref[...], preferred_element_type=jnp.float32) + b1_ref[...]
    g = jnp.maximum(h, 0.0) * me
    o_ref[...] = (jnp.dot(g, W2_ref[...], preferred_element_type=jnp.float32)
                  + jnp.dot(m, b2_ref[...], preferred_element_type=jnp.float32))


def kernel(x, node_weights, node_biases, w1s, b1s, w2s, b2s):
    B = x.shape[0]
    P, N, E = _routing_mats()
    nwT = node_weights.T
    nb = node_biases.reshape(1, N_NODES)
    W1 = jnp.transpose(w1s, (1, 0, 2)).reshape(IN_FEATURES, N_LEAVES * LEAF_WIDTH)
    b1 = b1s.reshape(1, N_LEAVES * LEAF_WIDTH)
    W2 = w2s.reshape(N_LEAVES * LEAF_WIDTH, OUT_FEATURES)

    grid = (B // BM,)
    full = lambda a: pl.BlockSpec(a.shape, lambda i: (0,) * a.ndim)
    out = pl.pallas_call(
        _fff_block,
        grid=grid,
        in_specs=[
            pl.BlockSpec((BM, IN_FEATURES), lambda i: (i, 0)),
            full(nwT), full(nb), full(P), full(N), full(E),
            full(W1), full(b1), full(W2), full(b2s),
        ],
        out_specs=pl.BlockSpec((BM, OUT_FEATURES), lambda i: (i, 0)),
        out_shape=jax.ShapeDtypeStruct((B, OUT_FEATURES), jnp.float32),
        compiler_params=pltpu.CompilerParams(
            dimension_semantics=("parallel",),
        ),
    )(x, nwT, nb, P, N, E, W1, b1, W2, b2s)
    return out


# speedup vs baseline: 1.2983x; 1.0653x over previous
import math

import jax
import jax.numpy as jnp
import numpy as np
from jax.experimental import pallas as pl
from jax.experimental.pallas import tpu as pltpu

IN_FEATURES = 1024
LEAF_WIDTH = 64
OUT_FEATURES = 1024
DEPTH = 4
N_LEAVES = 2 ** DEPTH
N_NODES = N_LEAVES - 1

BM = 512


def _routing_mats():
    P = np.zeros((N_NODES, N_LEAVES), np.float32)
    N = np.zeros((N_NODES, N_LEAVES), np.float32)
    for l in range(N_LEAVES):
        for d in range(DEPTH):
            g = l >> (DEPTH - 1 - d)
            node = (2 ** d - 1) + (g >> 1)
            if g & 1:
                P[node, l] = 1.0
            else:
                N[node, l] = 1.0
    E = np.kron(np.eye(N_LEAVES, dtype=np.float32), np.ones((1, LEAF_WIDTH), np.float32))
    return jnp.asarray(P), jnp.asarray(N), jnp.asarray(E)


def _fff_block(x_ref, nb_ref, P_ref, N_ref, E_ref,
               W1c_ref, b1_ref, W2_ref, b2_ref, o_ref):
    xb = x_ref[...]
    hz = jnp.dot(xb, W1c_ref[...], preferred_element_type=jnp.float32)
    h = hz[:, :N_LEAVES * LEAF_WIDTH] + b1_ref[...]
    z = hz[:, N_LEAVES * LEAF_WIDTH:] + nb_ref[...]
    ls_p = jax.nn.log_sigmoid(z)
    ls_n = jax.nn.log_sigmoid(-z)
    m = jnp.exp(jnp.dot(ls_p, P_ref[...], preferred_element_type=jnp.float32)
                + jnp.dot(ls_n, N_ref[...], preferred_element_type=jnp.float32))
    me = jnp.dot(m, E_ref[...], preferred_element_type=jnp.float32)
    g = jnp.maximum(h, 0.0) * me
    o_ref[...] = (jnp.dot(g, W2_ref[...], preferred_element_type=jnp.float32)
                  + jnp.dot(m, b2_ref[...], preferred_element_type=jnp.float32))


def kernel(x, node_weights, node_biases, w1s, b1s, w2s, b2s):
    B = x.shape[0]
    P, N, E = _routing_mats()
    nb = node_biases.reshape(1, N_NODES)
    W1 = jnp.transpose(w1s, (1, 0, 2)).reshape(IN_FEATURES, N_LEAVES * LEAF_WIDTH)
    W1c = jnp.concatenate([W1, node_weights.T], axis=1)
    b1 = b1s.reshape(1, N_LEAVES * LEAF_WIDTH)
    W2 = w2s.reshape(N_LEAVES * LEAF_WIDTH, OUT_FEATURES)

    grid = (B // BM,)
    full = lambda a: pl.BlockSpec(a.shape, lambda i: (0,) * a.ndim)
    out = pl.pallas_call(
        _fff_block,
        grid=grid,
        in_specs=[
            pl.BlockSpec((BM, IN_FEATURES), lambda i: (i, 0)),
            full(nb), full(P), full(N), full(E),
            full(W1c), full(b1), full(W2), full(b2s),
        ],
        out_specs=pl.BlockSpec((BM, OUT_FEATURES), lambda i: (i, 0)),
        out_shape=jax.ShapeDtypeStruct((B, OUT_FEATURES), jnp.float32),
        compiler_params=pltpu.CompilerParams(
            dimension_semantics=("parallel",),
        ),
    )(x, nb, P, N, E, W1c, b1, W2, b2s)
    return out


# speedup vs baseline: 1.3251x; 1.0207x over previous
import math

import jax
import jax.numpy as jnp
import numpy as np
from jax.experimental import pallas as pl
from jax.experimental.pallas import tpu as pltpu

IN_FEATURES = 1024
LEAF_WIDTH = 64
OUT_FEATURES = 1024
DEPTH = 4
N_LEAVES = 2 ** DEPTH
N_NODES = N_LEAVES - 1

BM = 1024


def _routing_mats():
    P = np.zeros((N_NODES, N_LEAVES), np.float32)
    N = np.zeros((N_NODES, N_LEAVES), np.float32)
    for l in range(N_LEAVES):
        for d in range(DEPTH):
            g = l >> (DEPTH - 1 - d)
            node = (2 ** d - 1) + (g >> 1)
            if g & 1:
                P[node, l] = 1.0
            else:
                N[node, l] = 1.0
    E = np.kron(np.eye(N_LEAVES, dtype=np.float32), np.ones((1, LEAF_WIDTH), np.float32))
    return jnp.asarray(P), jnp.asarray(N), jnp.asarray(E)


def _fff_block(x_ref, nb_ref, P_ref, N_ref, E_ref,
               W1c_ref, b1_ref, W2_ref, b2_ref, o_ref):
    xb = x_ref[...]
    hz = jnp.dot(xb, W1c_ref[...], preferred_element_type=jnp.float32)
    h = hz[:, :N_LEAVES * LEAF_WIDTH] + b1_ref[...]
    z = hz[:, N_LEAVES * LEAF_WIDTH:] + nb_ref[...]
    ls_p = jax.nn.log_sigmoid(z)
    ls_n = jax.nn.log_sigmoid(-z)
    m = jnp.exp(jnp.dot(ls_p, P_ref[...], preferred_element_type=jnp.float32)
                + jnp.dot(ls_n, N_ref[...], preferred_element_type=jnp.float32))
    me = jnp.dot(m, E_ref[...], preferred_element_type=jnp.float32)
    g = jnp.maximum(h, 0.0) * me
    o_ref[...] = (jnp.dot(g, W2_ref[...], preferred_element_type=jnp.float32)
                  + jnp.dot(m, b2_ref[...], preferred_element_type=jnp.float32))


def kernel(x, node_weights, node_biases, w1s, b1s, w2s, b2s):
    B = x.shape[0]
    P, N, E = _routing_mats()
    nb = node_biases.reshape(1, N_NODES)
    W1 = jnp.transpose(w1s, (1, 0, 2)).reshape(IN_FEATURES, N_LEAVES * LEAF_WIDTH)
    W1c = jnp.concatenate([W1, node_weights.T], axis=1)
    b1 = b1s.reshape(1, N_LEAVES * LEAF_WIDTH)
    W2 = w2s.reshape(N_LEAVES * LEAF_WIDTH, OUT_FEATURES)

    grid = (B // BM,)
    full = lambda a: pl.BlockSpec(a.shape, lambda i: (0,) * a.ndim)
    out = pl.pallas_call(
        _fff_block,
        grid=grid,
        in_specs=[
            pl.BlockSpec((BM, IN_FEATURES), lambda i: (i, 0)),
            full(nb), full(P), full(N), full(E),
            full(W1c), full(b1), full(W2), full(b2s),
        ],
        out_specs=pl.BlockSpec((BM, OUT_FEATURES), lambda i: (i, 0)),
        out_shape=jax.ShapeDtypeStruct((B, OUT_FEATURES), jnp.float32),
        compiler_params=pltpu.CompilerParams(
            dimension_semantics=("parallel",),
        ),
    )(x, nb, P, N, E, W1c, b1, W2, b2s)
    return out
